# baseline (speedup 1.0000x reference)
import jax
import jax.numpy as jnp
from jax import lax
from jax.experimental import pallas as pl
from jax.experimental.pallas import tpu as pltpu
from jax.experimental.pallas import tpu_sc as plsc

VOCAB = 100000
HIDDEN = 768
MAX_POS = 2048
BATCH = 4
SEQ = 2048
EPS = 1e-12

NC = 2
NS = 16
NW = NC * NS
TOK = BATCH * SEQ
NSL = 2
TOKS = TOK // NSL
TW = TOKS // NW
CR = 64
NCH = TW // CR
NBUF = 2


def _sc_body(word_hbm, ids_hbm, gath_hbm, idx_v, rows, gsem, wsem):
    c_ax = lax.axis_index("c")
    s_ax = lax.axis_index("s")
    wid = s_ax * NC + c_ax
    base = wid * TW

    pltpu.sync_copy(ids_hbm.at[wid], idx_v)

    def gather(c):
        return pltpu.async_copy(
            word_hbm.at[idx_v.at[c]], rows.at[c % NBUF], gsem.at[c % NBUF])

    def writeout(c):
        return pltpu.async_copy(
            rows.at[c % NBUF],
            gath_hbm.at[pl.ds(base + c * CR, CR)], wsem.at[c % NBUF])

    g = {0: gather(0), 1: gather(1)}
    w = {}
    for c in range(NCH):
        g[c].wait()
        w[c] = writeout(c)
        if c + NBUF < NCH:
            w[c].wait()
            g[c + NBUF] = gather(c + NBUF)
    for c in range(NCH - NBUF, NCH):
        w[c].wait()


@jax.jit
def _sc_gather(word_emb, ids_r):
    mesh = plsc.VectorSubcoreMesh(
        core_axis_name="c", subcore_axis_name="s",
        num_cores=NC, num_subcores=NS)
    return pl.kernel(
        _sc_body,
        out_type=jax.ShapeDtypeStruct((TOKS, HIDDEN), jnp.float32),
        mesh=mesh,
        scratch_types=[
            pltpu.VMEM((NCH, CR), jnp.int32),
            pltpu.VMEM((NBUF, CR, HIDDEN), jnp.float32),
            pltpu.SemaphoreType.DMA((NBUF,)),
            pltpu.SemaphoreType.DMA((NBUF,)),
        ],
    )(word_emb, ids_r)


RB = SEQ
NSTEP = TOKS // RB


def _tc_body(gath_ref, pos_ref, ttf_ref, typ_ref, lnw_ref, lnb_ref, out_ref):
    x = gath_ref[...]
    t0 = typ_ref[0, :]
    tdiff = typ_ref[1, :] - t0
    ttf = ttf_ref[0, 0, :]
    x = x + pos_ref[...] + t0[None, :] + ttf[:, None] * tdiff[None, :]
    s1 = jnp.sum(x, axis=-1, keepdims=True)
    s2 = jnp.sum(x * x, axis=-1, keepdims=True)
    mean = s1 * (1.0 / HIDDEN)
    var = s2 * (1.0 / HIDDEN) - mean * mean
    r = lax.rsqrt(var + EPS)
    out_ref[...] = ((x - mean) * r) * lnw_ref[0, :][None, :] \
        + lnb_ref[0, :][None, :]


@jax.jit
def _tc_add_ln(gathered, pos_emb, ttf, type_emb, lnw2, lnb2):
    return pl.pallas_call(
        _tc_body,
        grid=(NSTEP,),
        in_specs=[
            pl.BlockSpec((RB, HIDDEN), lambda i: (i, 0)),
            pl.BlockSpec((RB, HIDDEN), lambda i: (0, 0)),
            pl.BlockSpec((1, 1, RB), lambda i: (i, 0, 0)),
            pl.BlockSpec((2, HIDDEN), lambda i: (0, 0)),
            pl.BlockSpec((1, HIDDEN), lambda i: (0, 0)),
            pl.BlockSpec((1, HIDDEN), lambda i: (0, 0)),
        ],
        out_specs=pl.BlockSpec((RB, HIDDEN), lambda i: (i, 0)),
        out_shape=jax.ShapeDtypeStruct((TOKS, HIDDEN), jnp.float32),
    )(gathered, pos_emb, ttf, type_emb, lnw2, lnb2)


def kernel(input_ids, token_type_ids, word_emb, pos_emb, type_emb,
           ln_weight, ln_bias):
    ids_s = input_ids.astype(jnp.int32).reshape(NSL, NW, NCH, CR)
    ttf_s = token_type_ids.astype(jnp.float32).reshape(NSL, NSTEP, 1, RB)
    lnw2 = ln_weight.reshape(1, HIDDEN)
    lnb2 = ln_bias.reshape(1, HIDDEN)
    gath = [_sc_gather(word_emb, ids_s[k]) for k in range(NSL)]
    outs = [_tc_add_ln(gath[k], pos_emb, ttf_s[k], type_emb, lnw2, lnb2)
            for k in range(NSL)]
    out = jnp.concatenate(outs, axis=0)
    return out.reshape(BATCH, SEQ, HIDDEN)


# speedup vs baseline: 1.2822x; 1.2822x over previous
import jax
import jax.numpy as jnp
from jax import lax
from jax.experimental import pallas as pl
from jax.experimental.pallas import tpu as pltpu
from jax.experimental.pallas import tpu_sc as plsc

VOCAB = 100000
HIDDEN = 768
MAX_POS = 2048
BATCH = 4
SEQ = 2048
EPS = 1e-12

NC = 2
NS = 16
NW = NC * NS
TOK = BATCH * SEQ
NSL = 1
TOKS = TOK // NSL
TW = TOKS // NW
CR = 64
NCH = TW // CR
NBUF = 2


def _sc_body(word_hbm, ids_hbm, gath_hbm, idx_v, rows, gsem, wsem):
    c_ax = lax.axis_index("c")
    s_ax = lax.axis_index("s")
    wid = s_ax * NC + c_ax
    base = wid * TW

    pltpu.sync_copy(ids_hbm.at[wid], idx_v)

    def gather(c):
        return pltpu.async_copy(
            word_hbm.at[idx_v.at[c]], rows.at[c % NBUF], gsem.at[c % NBUF])

    def writeout(c):
        return pltpu.async_copy(
            rows.at[c % NBUF],
            gath_hbm.at[pl.ds(base + c * CR, CR)], wsem.at[c % NBUF])

    g = {0: gather(0), 1: gather(1)}
    w = {}
    for c in range(NCH):
        g[c].wait()
        w[c] = writeout(c)
        if c + NBUF < NCH:
            w[c].wait()
            g[c + NBUF] = gather(c + NBUF)
    for c in range(NCH - NBUF, NCH):
        w[c].wait()


@jax.jit
def _sc_gather(word_emb, ids_r):
    mesh = plsc.VectorSubcoreMesh(
        core_axis_name="c", subcore_axis_name="s",
        num_cores=NC, num_subcores=NS)
    return pl.kernel(
        _sc_body,
        out_type=jax.ShapeDtypeStruct((TOKS, HIDDEN), jnp.float32),
        mesh=mesh,
        scratch_types=[
            pltpu.VMEM((NCH, CR), jnp.int32),
            pltpu.VMEM((NBUF, CR, HIDDEN), jnp.float32),
            pltpu.SemaphoreType.DMA((NBUF,)),
            pltpu.SemaphoreType.DMA((NBUF,)),
        ],
    )(word_emb, ids_r)


RB = SEQ
NSTEP = TOKS // RB


def _tc_body(gath_ref, pos_ref, ttf_ref, typ_ref, lnw_ref, lnb_ref, out_ref):
    x = gath_ref[...]
    t0 = typ_ref[0, :]
    tdiff = typ_ref[1, :] - t0
    ttf = ttf_ref[0, 0, :]
    x = x + pos_ref[...] + t0[None, :] + ttf[:, None] * tdiff[None, :]
    s1 = jnp.sum(x, axis=-1, keepdims=True)
    s2 = jnp.sum(x * x, axis=-1, keepdims=True)
    mean = s1 * (1.0 / HIDDEN)
    var = s2 * (1.0 / HIDDEN) - mean * mean
    r = lax.rsqrt(var + EPS)
    out_ref[...] = ((x - mean) * r) * lnw_ref[0, :][None, :] \
        + lnb_ref[0, :][None, :]


@jax.jit
def _tc_add_ln(gathered, pos_emb, ttf, type_emb, lnw2, lnb2):
    return pl.pallas_call(
        _tc_body,
        grid=(NSTEP,),
        in_specs=[
            pl.BlockSpec((RB, HIDDEN), lambda i: (i, 0)),
            pl.BlockSpec((RB, HIDDEN), lambda i: (0, 0)),
            pl.BlockSpec((1, 1, RB), lambda i: (i, 0, 0)),
            pl.BlockSpec((2, HIDDEN), lambda i: (0, 0)),
            pl.BlockSpec((1, HIDDEN), lambda i: (0, 0)),
            pl.BlockSpec((1, HIDDEN), lambda i: (0, 0)),
        ],
        out_specs=pl.BlockSpec((RB, HIDDEN), lambda i: (i, 0)),
        out_shape=jax.ShapeDtypeStruct((TOKS, HIDDEN), jnp.float32),
    )(gathered, pos_emb, ttf, type_emb, lnw2, lnb2)


def kernel(input_ids, token_type_ids, word_emb, pos_emb, type_emb,
           ln_weight, ln_bias):
    ids_s = input_ids.astype(jnp.int32).reshape(NSL, NW, NCH, CR)
    ttf_s = token_type_ids.astype(jnp.float32).reshape(NSL, NSTEP, 1, RB)
    lnw2 = ln_weight.reshape(1, HIDDEN)
    lnb2 = ln_bias.reshape(1, HIDDEN)
    gath = [_sc_gather(word_emb, ids_s[k]) for k in range(NSL)]
    outs = [_tc_add_ln(gath[k], pos_emb, ttf_s[k], type_emb, lnw2, lnb2)
            for k in range(NSL)]
    out = jnp.concatenate(outs, axis=0)
    return out.reshape(BATCH, SEQ, HIDDEN)
